# baseline (speedup 1.0000x reference)
import jax
import jax.numpy as jnp
from jax.experimental import pallas as pl
from jax.experimental.pallas import tpu as pltpu

_K = 3
_DIL = 4
_PAD = _DIL * (_K - 1)


def _deinterleave_perm(kdim, csize):
    r = jax.lax.broadcasted_iota(jnp.int32, (kdim, kdim), 0)
    c = jax.lax.broadcasted_iota(jnp.int32, (kdim, kdim), 1)
    j = c // csize
    ci = c - j * csize
    return (r == ci * _K + j).astype(jnp.bfloat16)


def _fused_block_kernel(x_ref, v1_ref, g1_ref, b1_ref, v2_ref, g2_ref,
                        b2_ref, wd_ref, bd_ref, o_ref,
                        w1s_ref, w2s_ref, wds_ref, xc_ref, hc_ref):
    l_out = o_ref.shape[-1]
    c_in = x_ref.shape[1]
    c_out = o_ref.shape[1]
    n_rows = x_ref.shape[0]

    @pl.when(pl.program_id(0) == 0)
    def _prep():
        def prep(v_ref, g_ref, csize):
            v = v_ref[...]
            norm = jnp.sqrt(jnp.sum(v * v, axis=1, keepdims=True))
            wi = ((g_ref[...] / norm) * v).astype(jnp.bfloat16)
            p = _deinterleave_perm(v.shape[1], csize)
            return jnp.dot(wi, p,
                           preferred_element_type=jnp.float32
                           ).astype(jnp.bfloat16)

        w1s_ref[...] = prep(v1_ref, g1_ref, c_in)
        w2s_ref[...] = prep(v2_ref, g2_ref, c_out)
        wds_ref[...] = wd_ref[...].astype(jnp.bfloat16)

    cb = 512 if l_out % 512 == 0 else l_out
    nt = l_out // cb

    for row in range(n_rows):
      xc_r = xc_ref.at[row]
      xb = x_ref[row].astype(jnp.bfloat16)
      for j in range(_K):
        shift = (_K - 1 - j) * _DIL
        r0 = j * c_in
        if shift:
            xc_r[r0:r0 + c_in, :shift] = jnp.zeros((c_in, shift),
                                                   jnp.bfloat16)
            xc_r[r0:r0 + c_in, shift:] = xb[:, :l_out - shift]
        else:
            xc_r[r0:r0 + c_in, :] = xb

    for row in range(n_rows):
      xc_r = xc_ref.at[row]
      hc_r = hc_ref.at[row]
      for t in range(nt):
        t0 = t * cb
        acc = jnp.dot(w1s_ref[...], xc_r[:, t0:t0 + cb],
                      preferred_element_type=jnp.float32)
        h = jnp.maximum(acc + b1_ref[...], 0.0).astype(jnp.bfloat16)
        for j in range(_K):
            shift = (_K - 1 - j) * _DIL
            r0 = j * c_out
            if t == 0 and shift:
                hc_r[r0:r0 + c_out, :shift] = jnp.zeros(
                    (c_out, shift), jnp.bfloat16)
            hi = min(t0 + shift + cb, l_out)
            hc_r[r0:r0 + c_out, t0 + shift:hi] = h[:, :hi - t0 - shift]

    for row in range(n_rows):
      xc_r = xc_ref.at[row]
      hc_r = hc_ref.at[row]
      for t in range(nt):
        t0 = t * cb
        acc2 = jnp.dot(w2s_ref[...], hc_r[:, t0:t0 + cb],
                       preferred_element_type=jnp.float32)
        out = jnp.maximum(acc2 + b2_ref[...], 0.0)
        res = jnp.dot(wds_ref[...], xc_r[(_K - 1) * c_in:, t0:t0 + cb],
                      preferred_element_type=jnp.float32) + bd_ref[...]
        o_ref[row, :, t0:t0 + cb] = jnp.maximum(out + res, 0.0)


def kernel(x, v1, g1, b1, v2, g2, b2, wd, bd):
    n, c_in, l = x.shape
    c_out = v1.shape[0]

    v1r = v1.reshape(c_out, _K * c_in)
    v2r = v2.reshape(c_out, _K * c_out)
    wdr = wd.reshape(c_out, c_in)
    g1c = g1.reshape(c_out, 1)
    g2c = g2.reshape(c_out, 1)
    b1c = b1.reshape(c_out, 1)
    b2c = b2.reshape(c_out, 1)
    bdc = bd.reshape(c_out, 1)

    nb = 2 if n % 2 == 0 else 1
    return pl.pallas_call(
        _fused_block_kernel,
        out_shape=jax.ShapeDtypeStruct((n, c_out, l), x.dtype),
        grid_spec=pltpu.PrefetchScalarGridSpec(
            num_scalar_prefetch=0,
            grid=(n // nb,),
            in_specs=[
                pl.BlockSpec((nb, c_in, l), lambda b_: (b_, 0, 0)),
                pl.BlockSpec((c_out, _K * c_in), lambda b_: (0, 0)),
                pl.BlockSpec((c_out, 1), lambda b_: (0, 0)),
                pl.BlockSpec((c_out, 1), lambda b_: (0, 0)),
                pl.BlockSpec((c_out, _K * c_out), lambda b_: (0, 0)),
                pl.BlockSpec((c_out, 1), lambda b_: (0, 0)),
                pl.BlockSpec((c_out, 1), lambda b_: (0, 0)),
                pl.BlockSpec((c_out, c_in), lambda b_: (0, 0)),
                pl.BlockSpec((c_out, 1), lambda b_: (0, 0)),
            ],
            out_specs=pl.BlockSpec((nb, c_out, l), lambda b_: (b_, 0, 0)),
            scratch_shapes=[pltpu.VMEM((c_out, _K * c_in), jnp.bfloat16),
                            pltpu.VMEM((c_out, _K * c_out), jnp.bfloat16),
                            pltpu.VMEM((c_out, c_in), jnp.bfloat16),
                            pltpu.VMEM((nb, _K * c_in, l), jnp.bfloat16),
                            pltpu.VMEM((nb, _K * c_out, l), jnp.bfloat16)],
        ),
        compiler_params=pltpu.CompilerParams(
            dimension_semantics=("arbitrary",)),
    )(x, v1r, g1c, b1c, v2r, g2c, b2c, wdr, bdc)


# speedup vs baseline: 1.0428x; 1.0428x over previous
import jax
import jax.numpy as jnp
from jax.experimental import pallas as pl
from jax.experimental.pallas import tpu as pltpu

_K = 3
_DIL = 4
_PAD = _DIL * (_K - 1)


def _deinterleave_perm(kdim, csize):
    r = jax.lax.broadcasted_iota(jnp.int32, (kdim, kdim), 0)
    c = jax.lax.broadcasted_iota(jnp.int32, (kdim, kdim), 1)
    j = c // csize
    ci = c - j * csize
    return (r == ci * _K + j).astype(jnp.bfloat16)


def _fused_block_kernel(x_ref, v1_ref, g1_ref, b1_ref, v2_ref, g2_ref,
                        b2_ref, wd_ref, bd_ref, o_ref,
                        w1s_ref, w2s_ref, wds_ref, xc_ref, hc_ref):
    l_out = o_ref.shape[-1]
    c_in = x_ref.shape[1]
    c_out = o_ref.shape[1]
    n_rows = x_ref.shape[0]

    @pl.when(pl.program_id(0) == 0)
    def _prep():
        def prep(v_ref, g_ref, csize):
            v = v_ref[...]
            norm = jnp.sqrt(jnp.sum(v * v, axis=1, keepdims=True))
            wi = ((g_ref[...] / norm) * v).astype(jnp.bfloat16)
            p = _deinterleave_perm(v.shape[1], csize)
            return jnp.dot(wi, p,
                           preferred_element_type=jnp.float32
                           ).astype(jnp.bfloat16)

        w1s_ref[...] = prep(v1_ref, g1_ref, c_in)
        w2s_ref[...] = prep(v2_ref, g2_ref, c_out)
        wds_ref[...] = wd_ref[...].astype(jnp.bfloat16)

    cb = 1024 if l_out % 1024 == 0 else l_out
    nt = l_out // cb

    for row in range(n_rows):
      xc_r = xc_ref.at[row]
      xb = x_ref[row].astype(jnp.bfloat16)
      for j in range(_K):
        shift = (_K - 1 - j) * _DIL
        r0 = j * c_in
        if shift:
            xc_r[r0:r0 + c_in, :shift] = jnp.zeros((c_in, shift),
                                                   jnp.bfloat16)
            xc_r[r0:r0 + c_in, shift:] = xb[:, :l_out - shift]
        else:
            xc_r[r0:r0 + c_in, :] = xb

    for t in range(nt):
      for row in range(n_rows):
        xc_r = xc_ref.at[row]
        hc_r = hc_ref.at[row]
        t0 = t * cb
        acc = jnp.dot(w1s_ref[...], xc_r[:, t0:t0 + cb],
                      preferred_element_type=jnp.float32)
        h = jnp.maximum(acc + b1_ref[...], 0.0).astype(jnp.bfloat16)
        for j in range(_K):
            shift = (_K - 1 - j) * _DIL
            r0 = j * c_out
            if t == 0 and shift:
                hc_r[r0:r0 + c_out, :shift] = jnp.zeros(
                    (c_out, shift), jnp.bfloat16)
            hi = min(t0 + shift + cb, l_out)
            hc_r[r0:r0 + c_out, t0 + shift:hi] = h[:, :hi - t0 - shift]

    for t in range(nt):
      for row in range(n_rows):
        xc_r = xc_ref.at[row]
        hc_r = hc_ref.at[row]
        t0 = t * cb
        acc2 = jnp.dot(w2s_ref[...], hc_r[:, t0:t0 + cb],
                       preferred_element_type=jnp.float32)
        out = jnp.maximum(acc2 + b2_ref[...], 0.0)
        res = jnp.dot(wds_ref[...], xc_r[(_K - 1) * c_in:, t0:t0 + cb],
                      preferred_element_type=jnp.float32) + bd_ref[...]
        o_ref[row, :, t0:t0 + cb] = jnp.maximum(out + res, 0.0)


def kernel(x, v1, g1, b1, v2, g2, b2, wd, bd):
    n, c_in, l = x.shape
    c_out = v1.shape[0]

    v1r = v1.reshape(c_out, _K * c_in)
    v2r = v2.reshape(c_out, _K * c_out)
    wdr = wd.reshape(c_out, c_in)
    g1c = g1.reshape(c_out, 1)
    g2c = g2.reshape(c_out, 1)
    b1c = b1.reshape(c_out, 1)
    b2c = b2.reshape(c_out, 1)
    bdc = bd.reshape(c_out, 1)

    nb = 2 if n % 2 == 0 else 1
    return pl.pallas_call(
        _fused_block_kernel,
        out_shape=jax.ShapeDtypeStruct((n, c_out, l), x.dtype),
        grid_spec=pltpu.PrefetchScalarGridSpec(
            num_scalar_prefetch=0,
            grid=(n // nb,),
            in_specs=[
                pl.BlockSpec((nb, c_in, l), lambda b_: (b_, 0, 0)),
                pl.BlockSpec((c_out, _K * c_in), lambda b_: (0, 0)),
                pl.BlockSpec((c_out, 1), lambda b_: (0, 0)),
                pl.BlockSpec((c_out, 1), lambda b_: (0, 0)),
                pl.BlockSpec((c_out, _K * c_out), lambda b_: (0, 0)),
                pl.BlockSpec((c_out, 1), lambda b_: (0, 0)),
                pl.BlockSpec((c_out, 1), lambda b_: (0, 0)),
                pl.BlockSpec((c_out, c_in), lambda b_: (0, 0)),
                pl.BlockSpec((c_out, 1), lambda b_: (0, 0)),
            ],
            out_specs=pl.BlockSpec((nb, c_out, l), lambda b_: (b_, 0, 0)),
            scratch_shapes=[pltpu.VMEM((c_out, _K * c_in), jnp.bfloat16),
                            pltpu.VMEM((c_out, _K * c_out), jnp.bfloat16),
                            pltpu.VMEM((c_out, c_in), jnp.bfloat16),
                            pltpu.VMEM((nb, _K * c_in, l), jnp.bfloat16),
                            pltpu.VMEM((nb, _K * c_out, l), jnp.bfloat16)],
        ),
        compiler_params=pltpu.CompilerParams(
            dimension_semantics=("arbitrary",)),
    )(x, v1r, g1c, b1c, v2r, g2c, b2c, wdr, bdc)
